# Initial kernel scaffold; baseline (speedup 1.0000x reference)
import jax
import jax.numpy as jnp
from jax.experimental import pallas as pl


def kernel(vis_fs, lan_fs):
    raise NotImplementedError("write your pallas kernel here")


# speedup vs baseline: 696.5958x; 696.5958x over previous
import functools

import jax
import jax.numpy as jnp
from jax import lax
from jax.experimental import pallas as pl
from jax.experimental.pallas import tpu as pltpu
from jax.experimental.pallas import tpu_sc as plsc

BS = 32
QN = 32
FD = 256
NROW = BS * QN
NSEL = 4
LANES = 16
NEG = (BS - 1) * NSEL
LOGN = 128
NEG_FILL = -1e30


def _prep_body(v_ref, l_ref, u_ref, sim_ref):
    V = v_ref[...]
    L = l_ref[...]
    n2 = jnp.sum(V * V, axis=1, keepdims=True)
    nrm = jnp.maximum(jnp.sqrt(n2), 1e-12)
    u_ref[...] = V / nrm
    sim_ref[...] = lax.dot_general(L, V, (((1,), (1,)), ((), ())),
                                   preferred_element_type=jnp.float32)


def _prep(V, L):
    return pl.pallas_call(
        _prep_body,
        out_shape=[
            jax.ShapeDtypeStruct((NROW, FD), jnp.float32),
            jax.ShapeDtypeStruct((BS, NROW), jnp.float32),
        ],
    )(V, L)


def _gram_body(u_ref, g_ref):
    U = u_ref[...]
    g_ref[...] = lax.dot_general(U, U, (((1,), (1,)), ((), ())),
                                 preferred_element_type=jnp.float32)


def _gram(Uh32):
    return pl.pallas_call(
        _gram_body,
        out_shape=jax.ShapeDtypeStruct((NROW, NROW), jnp.float32),
    )(Uh32)


def _sc_mine(G, sim):
    info = plsc.get_sparse_core_info()
    nc = info.num_cores

    mesh = plsc.VectorSubcoreMesh(core_axis_name="c", subcore_axis_name="s")

    @functools.partial(
        pl.kernel,
        mesh=mesh,
        compiler_params=pltpu.CompilerParams(needs_layout_passes=False),
        out_type=jax.ShapeDtypeStruct((BS, LOGN), jnp.float32),
        scratch_types=[
            pltpu.VMEM((NROW,), jnp.float32),
            pltpu.VMEM((NROW,), jnp.float32),
            pltpu.VMEM((NROW,), jnp.float32),
            pltpu.VMEM((BS,), jnp.int32),
            pltpu.VMEM((NSEL, BS), jnp.int32),
            pltpu.VMEM((BS, NROW), jnp.float32),
            pltpu.VMEM((LOGN,), jnp.float32),
            pltpu.SemaphoreType.DMA,
        ],
    )
    def body(g_hbm, sim_hbm, out_hbm, sim_v, diff_v, uniq_v, selidx, selhist,
             gbuf, logits_v, sem):
        b = lax.axis_index("s") * nc + lax.axis_index("c")
        iota = lax.iota(jnp.int32, LANES)
        ones = jnp.ones((LANES,), jnp.float32)

        pltpu.sync_copy(sim_hbm.at[b], sim_v)

        def init_a(a, carry):
            base = a * QN
            s0 = sim_v[pl.ds(base, LANES)]
            s1 = sim_v[pl.ds(base + LANES, LANES)]
            mn = jnp.minimum(jnp.min(s0), jnp.min(s1))
            mx = jnp.maximum(jnp.max(s0), jnp.max(s1))
            den = mx - mn
            diff_v[pl.ds(base, LANES)] = (s0 - mn) / den
            diff_v[pl.ds(base + LANES, LANES)] = (s1 - mn) / den
            uniq_v[pl.ds(base, LANES)] = ones
            uniq_v[pl.ds(base + LANES, LANES)] = ones
            return carry

        lax.fori_loop(0, BS, init_a, 0)

        for it in range(NSEL):
            def sel_a(a, carry):
                sv0, sv1, first = carry
                base = a * QN
                s0 = uniq_v[pl.ds(base, LANES)] * diff_v[pl.ds(base, LANES)]
                s1 = (uniq_v[pl.ds(base + LANES, LANES)]
                      * diff_v[pl.ds(base + LANES, LANES)])
                m = jnp.maximum(jnp.max(s0), jnp.max(s1))
                big = jnp.int32(9999)
                q0 = jnp.min(jnp.where(s0 == m, iota, big))
                q1 = jnp.min(jnp.where(s1 == m, iota + LANES, big))
                q = jnp.minimum(q0, q1)
                sel = base + q
                upd = a != b
                sv0 = jnp.where(jnp.logical_and(iota == a, upd), sel, sv0)
                sv1 = jnp.where(jnp.logical_and(iota == a - LANES, upd), sel, sv1)
                first = jnp.where(jnp.logical_and(upd, first < 0), sel, first)
                return sv0, sv1, first

            z16 = jnp.zeros((LANES,), jnp.int32)
            sv0, sv1, first = lax.fori_loop(0, BS, sel_a,
                                            (z16, z16, jnp.int32(-1)))
            sv0 = jnp.where(iota == b, first, sv0)
            sv1 = jnp.where(iota == b - LANES, first, sv1)
            selidx[pl.ds(0, LANES)] = sv0
            selidx[pl.ds(LANES, LANES)] = sv1
            selhist[it, pl.ds(0, LANES)] = sv0
            selhist[it, pl.ds(LANES, LANES)] = sv1

            pltpu.async_copy(g_hbm.at[selidx], gbuf, sem).wait()

            def upd_v(v, carry):
                sl = pl.ds(v * LANES, LANES)

                def jmax(j, m):
                    return jnp.maximum(m, gbuf[j, sl])

                m = lax.fori_loop(1, BS, jmax, gbuf[0, sl])
                uniq_v[sl] = jnp.minimum(uniq_v[sl], (1.0 - m) * 0.5)
                return carry

            lax.fori_loop(0, NROW // LANES, upd_v, 0)

        fill = jnp.full((LANES,), NEG_FILL, jnp.float32)
        for c in range(LOGN // LANES):
            logits_v[pl.ds(c * LANES, LANES)] = fill

        for it in range(NSEL):
            for h in range(2):
                nvec = iota + h * LANES
                live = nvec < BS - 1
                avec = jnp.minimum(nvec + (nvec >= b).astype(jnp.int32),
                                   jnp.int32(BS - 1))
                rows = plsc.load_gather(
                    selhist, [jnp.full((LANES,), it, jnp.int32), avec],
                    mask=live)
                rows = jnp.where(live, rows, 0)
                vals = plsc.load_gather(sim_v, [rows], mask=live)
                posn = jnp.where(live, 1 + nvec * NSEL + it, 0)
                plsc.store_scatter(logits_v, [posn], vals, mask=live)

        p0 = sim_v[pl.ds(b * QN, LANES)]
        p1 = sim_v[pl.ds(b * QN + LANES, LANES)]
        pos = jnp.maximum(jnp.max(p0), jnp.max(p1))
        l0 = logits_v[pl.ds(0, LANES)]
        logits_v[pl.ds(0, LANES)] = jnp.where(iota == 0, pos, l0)

        pltpu.sync_copy(logits_v, out_hbm.at[b])

    return body(G, sim)


def _loss_body(lg_ref, out_ref):
    lg = lg_ref[...]
    m = jnp.max(lg, axis=1, keepdims=True)
    s = jnp.sum(jnp.exp(lg - m), axis=1, keepdims=True)
    lse = m + jnp.log(s)
    logp0 = lg[:, 0:1] - lse
    out_ref[...] = jnp.full((1, 1), -jnp.mean(logp0), jnp.float32)


def _loss(logits):
    return pl.pallas_call(
        _loss_body,
        out_shape=jax.ShapeDtypeStruct((1, 1), jnp.float32),
    )(logits)


def kernel(vis_fs, lan_fs):
    V = vis_fs.reshape(NROW, FD)
    L = lan_fs.reshape(BS, FD)
    U, sim = _prep(V, L)
    Uh32 = U.astype(jnp.float16).astype(jnp.float32)
    G = _gram(Uh32).astype(jnp.float16).astype(jnp.float32)
    logits = _sc_mine(G, sim)
    return _loss(logits).reshape(())


# speedup vs baseline: 1239.6684x; 1.7796x over previous
import functools

import jax
import jax.numpy as jnp
from jax import lax
from jax.experimental import pallas as pl
from jax.experimental.pallas import tpu as pltpu
from jax.experimental.pallas import tpu_sc as plsc

BS = 32
QN = 32
FD = 256
NROW = BS * QN
NSEL = 4
LANES = 16
NEG = (BS - 1) * NSEL
LOGN = 128
NEG_FILL = -1e30


def _f16_roundtrip(x):
    c = jnp.float32(8193.0)
    m = jnp.float32(0.75)
    y = x * c
    hi = y - (y - x)
    lo = (x + m) - m
    return jnp.where(jnp.abs(x) >= jnp.float32(2.0 ** -14), hi, lo)


def _prep_body(v_ref, l_ref, g_ref, sim_ref):
    V = v_ref[...]
    L = l_ref[...]
    n2 = jnp.sum(V * V, axis=1, keepdims=True)
    nrm = jnp.maximum(jnp.sqrt(n2), 1e-12)
    Uh = _f16_roundtrip(V / nrm)
    G = lax.dot_general(Uh, Uh, (((1,), (1,)), ((), ())),
                        preferred_element_type=jnp.float32)
    g_ref[...] = _f16_roundtrip(G)
    sim_ref[...] = lax.dot_general(L, V, (((1,), (1,)), ((), ())),
                                   preferred_element_type=jnp.float32)


def _prep(V, L):
    return pl.pallas_call(
        _prep_body,
        out_shape=[
            jax.ShapeDtypeStruct((NROW, NROW), jnp.float32),
            jax.ShapeDtypeStruct((BS, NROW), jnp.float32),
        ],
    )(V, L)


def _sc_mine(G, sim):
    info = plsc.get_sparse_core_info()
    nc = info.num_cores

    mesh = plsc.VectorSubcoreMesh(core_axis_name="c", subcore_axis_name="s")

    @functools.partial(
        pl.kernel,
        mesh=mesh,
        compiler_params=pltpu.CompilerParams(needs_layout_passes=False),
        out_type=jax.ShapeDtypeStruct((BS, LOGN), jnp.float32),
        scratch_types=[
            pltpu.VMEM((NROW,), jnp.float32),
            pltpu.VMEM((NROW,), jnp.float32),
            pltpu.VMEM((NROW,), jnp.float32),
            pltpu.VMEM((BS,), jnp.int32),
            pltpu.VMEM((NSEL, BS), jnp.int32),
            pltpu.VMEM((BS, NROW), jnp.float32),
            pltpu.VMEM((LOGN,), jnp.float32),
            pltpu.SemaphoreType.DMA,
        ],
    )
    def body(g_hbm, sim_hbm, out_hbm, sim_v, diff_v, uniq_v, selidx, selhist,
             gbuf, logits_v, sem):
        b = lax.axis_index("s") * nc + lax.axis_index("c")
        iota = lax.iota(jnp.int32, LANES)
        ones = jnp.ones((LANES,), jnp.float32)

        pltpu.sync_copy(sim_hbm.at[b], sim_v)

        def init_a(a, carry):
            base = a * QN
            s0 = sim_v[pl.ds(base, LANES)]
            s1 = sim_v[pl.ds(base + LANES, LANES)]
            mn = jnp.minimum(jnp.min(s0), jnp.min(s1))
            mx = jnp.maximum(jnp.max(s0), jnp.max(s1))
            den = mx - mn
            diff_v[pl.ds(base, LANES)] = (s0 - mn) / den
            diff_v[pl.ds(base + LANES, LANES)] = (s1 - mn) / den
            uniq_v[pl.ds(base, LANES)] = ones
            uniq_v[pl.ds(base + LANES, LANES)] = ones
            return carry

        lax.fori_loop(0, BS, init_a, 0)

        for it in range(NSEL):
            def sel_a(a, carry):
                sv0, sv1, first = carry
                base = a * QN
                s0 = uniq_v[pl.ds(base, LANES)] * diff_v[pl.ds(base, LANES)]
                s1 = (uniq_v[pl.ds(base + LANES, LANES)]
                      * diff_v[pl.ds(base + LANES, LANES)])
                m = jnp.maximum(jnp.max(s0), jnp.max(s1))
                big = jnp.int32(9999)
                q0 = jnp.min(jnp.where(s0 == m, iota, big))
                q1 = jnp.min(jnp.where(s1 == m, iota + LANES, big))
                q = jnp.minimum(q0, q1)
                sel = base + q
                upd = a != b
                sv0 = jnp.where(jnp.logical_and(iota == a, upd), sel, sv0)
                sv1 = jnp.where(jnp.logical_and(iota == a - LANES, upd), sel, sv1)
                first = jnp.where(jnp.logical_and(upd, first < 0), sel, first)
                return sv0, sv1, first

            z16 = jnp.zeros((LANES,), jnp.int32)
            sv0, sv1, first = lax.fori_loop(0, BS, sel_a,
                                            (z16, z16, jnp.int32(-1)))
            sv0 = jnp.where(iota == b, first, sv0)
            sv1 = jnp.where(iota == b - LANES, first, sv1)
            selidx[pl.ds(0, LANES)] = sv0
            selidx[pl.ds(LANES, LANES)] = sv1
            selhist[it, pl.ds(0, LANES)] = sv0
            selhist[it, pl.ds(LANES, LANES)] = sv1

            pltpu.async_copy(g_hbm.at[selidx], gbuf, sem).wait()

            def upd_v(v, carry):
                sl = pl.ds(v * LANES, LANES)
                ms = [jnp.maximum(gbuf[2 * j, sl], gbuf[2 * j + 1, sl])
                      for j in range(BS // 2)]
                while len(ms) > 1:
                    ms = [jnp.maximum(ms[2 * j], ms[2 * j + 1])
                          for j in range(len(ms) // 2)]
                uniq_v[sl] = jnp.minimum(uniq_v[sl], (1.0 - ms[0]) * 0.5)
                return carry

            lax.fori_loop(0, NROW // LANES, upd_v, 0)

        fill = jnp.full((LANES,), NEG_FILL, jnp.float32)
        for c in range(LOGN // LANES):
            logits_v[pl.ds(c * LANES, LANES)] = fill

        for it in range(NSEL):
            for h in range(2):
                nvec = iota + h * LANES
                live = nvec < BS - 1
                avec = jnp.minimum(nvec + (nvec >= b).astype(jnp.int32),
                                   jnp.int32(BS - 1))
                rows = plsc.load_gather(
                    selhist, [jnp.full((LANES,), it, jnp.int32), avec],
                    mask=live)
                rows = jnp.where(live, rows, 0)
                vals = plsc.load_gather(sim_v, [rows], mask=live)
                posn = jnp.where(live, 1 + nvec * NSEL + it, 0)
                plsc.store_scatter(logits_v, [posn], vals, mask=live)

        p0 = sim_v[pl.ds(b * QN, LANES)]
        p1 = sim_v[pl.ds(b * QN + LANES, LANES)]
        pos = jnp.maximum(jnp.max(p0), jnp.max(p1))
        l0 = logits_v[pl.ds(0, LANES)]
        logits_v[pl.ds(0, LANES)] = jnp.where(iota == 0, pos, l0)

        pltpu.sync_copy(logits_v, out_hbm.at[b])

    return body(G, sim)


def _loss_body(lg_ref, out_ref):
    lg = lg_ref[...]
    m = jnp.max(lg, axis=1, keepdims=True)
    s = jnp.sum(jnp.exp(lg - m), axis=1, keepdims=True)
    lse = m + jnp.log(s)
    logp0 = lg[:, 0:1] - lse
    out_ref[...] = jnp.full((1, 1), -jnp.mean(logp0), jnp.float32)


def _loss(logits):
    return pl.pallas_call(
        _loss_body,
        out_shape=jax.ShapeDtypeStruct((1, 1), jnp.float32),
    )(logits)


def kernel(vis_fs, lan_fs):
    V = vis_fs.reshape(NROW, FD)
    L = lan_fs.reshape(BS, FD)
    G, sim = _prep(V, L)
    logits = _sc_mine(G, sim)
    return _loss(logits).reshape(())
